# baseline (device time: 103846 ns/iter reference)
import jax
import jax.numpy as jnp
from jax import lax
from jax.experimental import pallas as pl
from jax.experimental.pallas import tpu as pltpu

N_DEV = 16
HOPS = N_DEV // 2
XP = 4
WP = 8


def kernel(x, w_mat, scale_x, scale_w):
    m_per, k = x.shape
    n_per = w_mat.shape[1]
    half = m_per // 2
    xrows = m_per // XP
    wrows = w_mat.shape[0] // WP
    scale = (scale_x * scale_w).astype(jnp.float32).reshape(1, 1)

    def body(x_hbm, w_hbm, s_ref, out_hbm, x8, w8, gr_ref, gl_ref,
             sx, sw, band, sx_sems, sw_sems, band_sems,
             r_send, r_recv, l_send, l_recv):
        my = lax.axis_index("i")
        left = (my - 1) % N_DEV
        right = (my + 1) % N_DEV

        def stage(src_hbm, p, rows, buf, sems):
            return pltpu.make_async_copy(
                src_hbm.at[pl.ds(p * rows, rows), :], buf.at[p % 2], sems.at[p]
            )

        for p in range(min(2, XP)):
            stage(x_hbm, p, xrows, sx, sx_sems).start()

        barrier_sem = pltpu.get_barrier_semaphore()
        for nbr in (left, right):
            pl.semaphore_signal(
                barrier_sem, inc=1,
                device_id=(nbr,), device_id_type=pl.DeviceIdType.MESH,
            )
        pl.semaphore_wait(barrier_sem, 2)

        s = s_ref[0, 0]

        band_dmas = []

        def mm_band(i, row0, pieces):
            if i >= 2:
                band_dmas[i - 2].wait()
            for chunk, off, rows in pieces:
                acc = lax.dot_general(
                    chunk, w8[...],
                    (((1,), (0,)), ((), ())),
                    preferred_element_type=jnp.float32,
                )
                band[i % 2, pl.ds(off, rows), :] = jnp.maximum(acc * s, 0.0)
            d = pltpu.make_async_copy(
                band.at[i % 2], out_hbm.at[pl.ds(row0, m_per), :],
                band_sems.at[i % 2],
            )
            d.start()
            band_dmas.append(d)

        def mk(h, j, slots, send_sems, recv_sems, tgt):
            rows = pl.ds(j * half, half)
            src = x8 if h == 0 else slots.at[h - 1]
            return pltpu.make_async_remote_copy(
                src_ref=src.at[rows, :],
                dst_ref=slots.at[h, rows, :],
                send_sem=send_sems.at[h, j],
                recv_sem=recv_sems.at[h, j],
                device_id=(tgt,),
                device_id_type=pl.DeviceIdType.MESH,
            )

        def halves(h):
            if h < HOPS - 1:
                return (0, 1), (0, 1)
            return (0,), (1,)

        rs = {(h, j): mk(h, j, gr_ref, r_send, r_recv, right)
              for h in range(HOPS) for j in halves(h)[0]}
        ls = {(h, j): mk(h, j, gl_ref, l_send, l_recv, left)
              for h in range(HOPS) for j in halves(h)[1]}

        for p in range(XP):
            stage(x_hbm, p, xrows, sx, sx_sems).wait()
            x8[pl.ds(p * xrows, xrows), :] = sx[p % 2].astype(jnp.float8_e4m3fn)
            if p + 2 < XP:
                stage(x_hbm, p + 2, xrows, sx, sx_sems).start()
            if (p + 1) * xrows == half:
                rs[(0, 0)].start()
                ls[(0, 0)].start()
            elif (p + 1) * xrows == m_per:
                rs[(0, 1)].start()
                ls[(0, 1)].start()

        for p in range(min(2, WP)):
            stage(w_hbm, p, wrows, sw, sw_sems).start()
        for p in range(WP):
            stage(w_hbm, p, wrows, sw, sw_sems).wait()
            w8[pl.ds(p * wrows, wrows), :] = sw[p % 2].astype(jnp.float8_e4m3fn)
            if p + 2 < WP:
                stage(w_hbm, p + 2, wrows, sw, sw_sems).start()

        mm_band(0, my * m_per, [(x8[...], 0, m_per)])

        for h in range(HOPS):
            rj, lj = halves(h)
            for j in rj:
                rs[(h, j)].wait_recv()
                if (h + 1, j) in rs:
                    rs[(h + 1, j)].start()
            for j in lj:
                ls[(h, j)].wait_recv()
                if (h + 1, j) in ls:
                    ls[(h + 1, j)].start()
            if h < HOPS - 1:
                mm_band(1 + 2 * h, ((my - h - 1) % N_DEV) * m_per,
                        [(gr_ref[h], 0, m_per)])
                mm_band(2 + 2 * h, ((my + h + 1) % N_DEV) * m_per,
                        [(gl_ref[h], 0, m_per)])
            else:
                anti = (my + HOPS) % N_DEV
                mm_band(15, anti * m_per,
                        [(gr_ref[h, :half, :], 0, half),
                         (gl_ref[h, half:, :], half, half)])

        for r in list(rs.values()) + list(ls.values()):
            r.wait_send()
        band_dmas[14].wait()
        band_dmas[15].wait()

    fp8 = jnp.float8_e4m3fn
    return pl.pallas_call(
        body,
        out_shape=jax.ShapeDtypeStruct((N_DEV * m_per, n_per), jnp.float32),
        in_specs=[
            pl.BlockSpec(memory_space=pltpu.MemorySpace.HBM),
            pl.BlockSpec(memory_space=pltpu.MemorySpace.HBM),
            pl.BlockSpec(memory_space=pltpu.SMEM),
        ],
        out_specs=pl.BlockSpec(memory_space=pltpu.MemorySpace.HBM),
        scratch_shapes=[
            pltpu.VMEM((m_per, k), fp8),
            pltpu.VMEM((w_mat.shape[0], n_per), fp8),
            pltpu.VMEM((HOPS, m_per, k), fp8),
            pltpu.VMEM((HOPS, m_per, k), fp8),
            pltpu.VMEM((2, xrows, k), jnp.float32),
            pltpu.VMEM((2, wrows, n_per), jnp.float32),
            pltpu.VMEM((2, m_per, n_per), jnp.float32),
            pltpu.SemaphoreType.DMA((XP,)),
            pltpu.SemaphoreType.DMA((WP,)),
            pltpu.SemaphoreType.DMA((2,)),
            pltpu.SemaphoreType.DMA((HOPS, 2)),
            pltpu.SemaphoreType.DMA((HOPS, 2)),
            pltpu.SemaphoreType.DMA((HOPS, 2)),
            pltpu.SemaphoreType.DMA((HOPS, 2)),
        ],
        compiler_params=pltpu.CompilerParams(collective_id=0),
    )(x, w_mat, scale)


# device time: 101496 ns/iter; 1.0232x vs baseline; 1.0232x over previous
import jax
import jax.numpy as jnp
from jax import lax
from jax.experimental import pallas as pl
from jax.experimental.pallas import tpu as pltpu

N_DEV = 16
HOPS = N_DEV // 2
XP = 4
WP = 8


def kernel(x, w_mat, scale_x, scale_w):
    m_per, k = x.shape
    n_per = w_mat.shape[1]
    half = m_per // 2
    quart = half // 2
    xrows = m_per // XP
    wrows = w_mat.shape[0] // WP
    scale = (scale_x * scale_w).astype(jnp.float32).reshape(1, 1)

    def body(x_hbm, w_hbm, s_ref, out_hbm, x8, w8, gr_ref, gl_ref,
             sx, sw, band, sx_sems, sw_sems, band_sems,
             r_send, r_recv, l_send, l_recv):
        my = lax.axis_index("i")
        left = (my - 1) % N_DEV
        right = (my + 1) % N_DEV

        def stage(src_hbm, p, rows, buf, sems):
            return pltpu.make_async_copy(
                src_hbm.at[pl.ds(p * rows, rows), :], buf.at[p % 2], sems.at[p]
            )

        for p in range(min(2, XP)):
            stage(x_hbm, p, xrows, sx, sx_sems).start()

        barrier_sem = pltpu.get_barrier_semaphore()
        for nbr in (left, right):
            pl.semaphore_signal(
                barrier_sem, inc=1,
                device_id=(nbr,), device_id_type=pl.DeviceIdType.MESH,
            )
        pl.semaphore_wait(barrier_sem, 2)

        s = s_ref[0, 0]

        band_dmas = []

        def mm_band(i, row0, pieces):
            if i >= 2:
                band_dmas[i - 2].wait()
            for chunk, off, rows in pieces:
                acc = lax.dot_general(
                    chunk, w8[...],
                    (((1,), (0,)), ((), ())),
                    preferred_element_type=jnp.float32,
                )
                band[i % 2, pl.ds(off, rows), :] = jnp.maximum(acc * s, 0.0)
            d = pltpu.make_async_copy(
                band.at[i % 2], out_hbm.at[pl.ds(row0, m_per), :],
                band_sems.at[i % 2],
            )
            d.start()
            band_dmas.append(d)

        def mk(h, j, slots, send_sems, recv_sems, tgt, base):
            if h < HOPS - 1:
                rows = pl.ds(j * half, half)
            else:
                rows = pl.ds(base + j * quart, quart)
            src = x8 if h == 0 else slots.at[h - 1]
            return pltpu.make_async_remote_copy(
                src_ref=src.at[rows, :],
                dst_ref=slots.at[h, rows, :],
                send_sem=send_sems.at[h, j],
                recv_sem=recv_sems.at[h, j],
                device_id=(tgt,),
                device_id_type=pl.DeviceIdType.MESH,
            )

        rs = {(h, j): mk(h, j, gr_ref, r_send, r_recv, right, 0)
              for h in range(HOPS) for j in (0, 1)}
        ls = {(h, j): mk(h, j, gl_ref, l_send, l_recv, left, half)
              for h in range(HOPS) for j in (0, 1)}

        for p in range(XP):
            stage(x_hbm, p, xrows, sx, sx_sems).wait()
            x8[pl.ds(p * xrows, xrows), :] = sx[p % 2].astype(jnp.float8_e4m3fn)
            if p + 2 < XP:
                stage(x_hbm, p + 2, xrows, sx, sx_sems).start()
            if (p + 1) * xrows == half:
                rs[(0, 0)].start()
                ls[(0, 0)].start()
            elif (p + 1) * xrows == m_per:
                rs[(0, 1)].start()
                ls[(0, 1)].start()

        for p in range(min(2, WP)):
            stage(w_hbm, p, wrows, sw, sw_sems).start()
        for p in range(WP):
            stage(w_hbm, p, wrows, sw, sw_sems).wait()
            w8[pl.ds(p * wrows, wrows), :] = sw[p % 2].astype(jnp.float8_e4m3fn)
            if p + 2 < WP:
                stage(w_hbm, p + 2, wrows, sw, sw_sems).start()

        mm_band(0, my * m_per, [(x8[...], 0, m_per)])

        for h in range(HOPS):
            for j in (0, 1):
                for flow, qj in ((rs, 0), (ls, 1)):
                    flow[(h, j)].wait_recv()
                    if h + 1 < HOPS - 1:
                        flow[(h + 1, j)].start()
                    elif h + 1 == HOPS - 1 and j == qj:
                        flow[(HOPS - 1, 0)].start()
                        flow[(HOPS - 1, 1)].start()
            if h < HOPS - 1:
                mm_band(1 + 2 * h, ((my - h - 1) % N_DEV) * m_per,
                        [(gr_ref[h], 0, m_per)])
                mm_band(2 + 2 * h, ((my + h + 1) % N_DEV) * m_per,
                        [(gl_ref[h], 0, m_per)])
            else:
                anti = (my + HOPS) % N_DEV
                mm_band(15, anti * m_per,
                        [(gr_ref[h, :half, :], 0, half),
                         (gl_ref[h, half:, :], half, half)])

        for r in list(rs.values()) + list(ls.values()):
            r.wait_send()
        band_dmas[14].wait()
        band_dmas[15].wait()

    fp8 = jnp.float8_e4m3fn
    return pl.pallas_call(
        body,
        out_shape=jax.ShapeDtypeStruct((N_DEV * m_per, n_per), jnp.float32),
        in_specs=[
            pl.BlockSpec(memory_space=pltpu.MemorySpace.HBM),
            pl.BlockSpec(memory_space=pltpu.MemorySpace.HBM),
            pl.BlockSpec(memory_space=pltpu.SMEM),
        ],
        out_specs=pl.BlockSpec(memory_space=pltpu.MemorySpace.HBM),
        scratch_shapes=[
            pltpu.VMEM((m_per, k), fp8),
            pltpu.VMEM((w_mat.shape[0], n_per), fp8),
            pltpu.VMEM((HOPS, m_per, k), fp8),
            pltpu.VMEM((HOPS, m_per, k), fp8),
            pltpu.VMEM((2, xrows, k), jnp.float32),
            pltpu.VMEM((2, wrows, n_per), jnp.float32),
            pltpu.VMEM((2, m_per, n_per), jnp.float32),
            pltpu.SemaphoreType.DMA((XP,)),
            pltpu.SemaphoreType.DMA((WP,)),
            pltpu.SemaphoreType.DMA((2,)),
            pltpu.SemaphoreType.DMA((HOPS, 2)),
            pltpu.SemaphoreType.DMA((HOPS, 2)),
            pltpu.SemaphoreType.DMA((HOPS, 2)),
            pltpu.SemaphoreType.DMA((HOPS, 2)),
        ],
        compiler_params=pltpu.CompilerParams(collective_id=0),
    )(x, w_mat, scale)
